# baseline (speedup 1.0000x reference)
import functools

import jax
import jax.numpy as jnp
from jax import lax
from jax.experimental import pallas as pl
from jax.experimental.pallas import tpu as pltpu
from jax.experimental.pallas import tpu_sc as plsc

N = 256 * 24 * 325
NC, NS, L = 2, 16, 16
NW = NC * NS
PER_W = N // NW
CHUNK = 6240
NCHUNK = PER_W // CHUNK
VECS = CHUNK // L
UNROLL = 10
NACC = 4
PROW = 2 * L


def _lane_shuffle(x, idx):
    dnums = lax.GatherDimensionNumbers(
        offset_dims=(), collapsed_slice_dims=(0,), start_index_map=(0,))
    return lax.gather(x, idx[:, None], dimension_numbers=dnums,
                      slice_sizes=(1,),
                      mode=lax.GatherScatterMode.PROMISE_IN_BOUNDS)


def _lane_sum_all(x):
    iota = lax.iota(jnp.int32, L)
    for shift in (8, 4, 2, 1):
        x = x + _lane_shuffle(x, iota ^ shift)
    return x


def _mae_body(pred_hbm, true_hbm, out_hbm,
              pred0, pred1, true0, true1, partial_v, red_v, out_v,
              shared, sems):
    cid = lax.axis_index("c")
    sid = lax.axis_index("s")
    wid = sid * NC + cid
    base = wid * PER_W
    pred_bufs = (pred0, pred1)
    true_bufs = (true0, true1)

    def start(slot, j):
        off = base + j * CHUNK
        pltpu.make_async_copy(
            pred_hbm.at[pl.ds(off, CHUNK)], pred_bufs[slot],
            sems.at[slot, 0]).start()
        pltpu.make_async_copy(
            true_hbm.at[pl.ds(off, CHUNK)], true_bufs[slot],
            sems.at[slot, 1]).start()

    def wait(slot):
        pltpu.make_async_copy(
            pred_hbm.at[pl.ds(0, CHUNK)], pred_bufs[slot],
            sems.at[slot, 0]).wait()
        pltpu.make_async_copy(
            true_hbm.at[pl.ds(0, CHUNK)], true_bufs[slot],
            sems.at[slot, 1]).wait()

    start(0, 0)
    zero = jnp.zeros((L,), jnp.float32)
    accs = (zero,) * NACC + (zero,) * NACC
    for j in range(NCHUNK):
        slot = j % 2
        if j + 1 < NCHUNK:
            start(1 - slot, j + 1)
        wait(slot)
        pv, tv = pred_bufs[slot], true_bufs[slot]

        def vec_step(i, c, pv=pv, tv=tv):
            regs = list(c)
            for u in range(UNROLL):
                r = u % NACC
                p = pv[pl.ds((i + u) * L, L)]
                t = tv[pl.ds((i + u) * L, L)]
                m = t != 0.0
                regs[r] = regs[r] + jnp.where(m, jnp.abs(p - t), 0.0)
                regs[NACC + r] = regs[NACC + r] + jnp.where(m, 1.0, 0.0)
            return tuple(regs)

        accs = plsc.parallel_loop(0, VECS, step=UNROLL, carry=accs)(vec_step)

    acc = accs[0]
    cnt = accs[NACC]
    for r in range(1, NACC):
        acc = acc + accs[r]
        cnt = cnt + accs[NACC + r]

    partial_v[pl.ds(0, L)] = acc
    partial_v[pl.ds(L, L)] = cnt
    pltpu.sync_copy(partial_v, shared.at[pl.ds(sid * PROW, PROW)])
    plsc.subcore_barrier()

    @pl.when(sid == 0)
    def _():
        pltpu.sync_copy(shared, red_v)
        acc0, cnt0 = zero, zero
        for i in range(NS):
            acc0 = acc0 + red_v[pl.ds(i * PROW, L)]
            cnt0 = cnt0 + red_v[pl.ds(i * PROW + L, L)]
        out_v[pl.ds(0, L)] = _lane_sum_all(acc0)
        out_v[pl.ds(L, L)] = _lane_sum_all(cnt0)
        pltpu.sync_copy(out_v, out_hbm.at[pl.ds(cid * PROW, PROW)])


@jax.jit
def _mae_sc(pred_flat, true_flat):
    mesh = plsc.VectorSubcoreMesh(core_axis_name="c", subcore_axis_name="s")
    run = pl.kernel(
        _mae_body,
        out_type=jax.ShapeDtypeStruct((NC * PROW,), jnp.float32),
        mesh=mesh,
        scratch_types=[
            pltpu.VMEM((CHUNK,), jnp.float32),
            pltpu.VMEM((CHUNK,), jnp.float32),
            pltpu.VMEM((CHUNK,), jnp.float32),
            pltpu.VMEM((CHUNK,), jnp.float32),
            pltpu.VMEM((PROW,), jnp.float32),
            pltpu.VMEM((NS * PROW,), jnp.float32),
            pltpu.VMEM((PROW,), jnp.float32),
            pltpu.VMEM_SHARED((NS * PROW,), jnp.float32),
            pltpu.SemaphoreType.DMA((2, 2)),
        ],
    )
    return run(pred_flat, true_flat)


def kernel(y_pred, y_true):
    p = y_pred.transpose(1, 2, 3, 0).reshape(N)
    t = y_true.transpose(1, 2, 3, 0).reshape(N)
    out = _mae_sc(p, t)
    return (out[0] + out[PROW]) / (out[L] + out[PROW + L])


# speedup vs baseline: 1.0868x; 1.0868x over previous
import functools

import jax
import jax.numpy as jnp
from jax import lax
from jax.experimental import pallas as pl
from jax.experimental.pallas import tpu as pltpu
from jax.experimental.pallas import tpu_sc as plsc

N = 256 * 24 * 325
LANES = 128
ROWS = N // LANES

R_SC = 7800
N_SC = R_SC * LANES
BR = 600
TC_GRID = (ROWS - R_SC) // BR

NC, NS, L = 2, 16, 16
NW = NC * NS
PER_W = N_SC // NW
NCHUNK = 5
CHUNK = PER_W // NCHUNK
VECS = CHUNK // L
UNROLL = 10
NACC = 4
PROW = 2 * L

assert PER_W * NW == N_SC and CHUNK * NCHUNK == PER_W
assert CHUNK % L == 0 and VECS % UNROLL == 0 and R_SC % BR == 0


def _lane_shuffle(x, idx):
    dnums = lax.GatherDimensionNumbers(
        offset_dims=(), collapsed_slice_dims=(0,), start_index_map=(0,))
    return lax.gather(x, idx[:, None], dimension_numbers=dnums,
                      slice_sizes=(1,),
                      mode=lax.GatherScatterMode.PROMISE_IN_BOUNDS)


def _lane_sum_all(x):
    iota = lax.iota(jnp.int32, L)
    for shift in (8, 4, 2, 1):
        x = x + _lane_shuffle(x, iota ^ shift)
    return x


def _mae_sc_body(pred_hbm, true_hbm, out_hbm,
                 pred0, pred1, true0, true1, partial_v, red_v, out_v,
                 shared, sems):
    cid = lax.axis_index("c")
    sid = lax.axis_index("s")
    wid = sid * NC + cid
    base = wid * PER_W
    pred_bufs = (pred0, pred1)
    true_bufs = (true0, true1)

    def start(slot, j):
        off = base + j * CHUNK
        pltpu.make_async_copy(
            pred_hbm.at[pl.ds(off, CHUNK)], pred_bufs[slot],
            sems.at[slot, 0]).start()
        pltpu.make_async_copy(
            true_hbm.at[pl.ds(off, CHUNK)], true_bufs[slot],
            sems.at[slot, 1]).start()

    def wait(slot):
        pltpu.make_async_copy(
            pred_hbm.at[pl.ds(0, CHUNK)], pred_bufs[slot],
            sems.at[slot, 0]).wait()
        pltpu.make_async_copy(
            true_hbm.at[pl.ds(0, CHUNK)], true_bufs[slot],
            sems.at[slot, 1]).wait()

    start(0, 0)
    zero = jnp.zeros((L,), jnp.float32)
    accs = (zero,) * NACC + (zero,) * NACC
    for j in range(NCHUNK):
        slot = j % 2
        if j + 1 < NCHUNK:
            start(1 - slot, j + 1)
        wait(slot)
        pv, tv = pred_bufs[slot], true_bufs[slot]

        def vec_step(i, c, pv=pv, tv=tv):
            regs = list(c)
            for u in range(UNROLL):
                r = u % NACC
                p = pv[pl.ds((i + u) * L, L)]
                t = tv[pl.ds((i + u) * L, L)]
                m = t != 0.0
                regs[r] = regs[r] + jnp.where(m, jnp.abs(p - t), 0.0)
                regs[NACC + r] = regs[NACC + r] + jnp.where(m, 1.0, 0.0)
            return tuple(regs)

        accs = plsc.parallel_loop(0, VECS, step=UNROLL, carry=accs)(vec_step)

    acc = accs[0]
    cnt = accs[NACC]
    for r in range(1, NACC):
        acc = acc + accs[r]
        cnt = cnt + accs[NACC + r]

    partial_v[pl.ds(0, L)] = acc
    partial_v[pl.ds(L, L)] = cnt
    pltpu.sync_copy(partial_v, shared.at[pl.ds(sid * PROW, PROW)])
    plsc.subcore_barrier()

    @pl.when(sid == 0)
    def _():
        pltpu.sync_copy(shared, red_v)
        acc0, cnt0 = zero, zero
        for i in range(NS):
            acc0 = acc0 + red_v[pl.ds(i * PROW, L)]
            cnt0 = cnt0 + red_v[pl.ds(i * PROW + L, L)]
        out_v[pl.ds(0, L)] = _lane_sum_all(acc0)
        out_v[pl.ds(L, L)] = _lane_sum_all(cnt0)
        pltpu.sync_copy(out_v, out_hbm.at[pl.ds(cid * PROW, PROW)])


def _mae_sc(pred_flat, true_flat):
    mesh = plsc.VectorSubcoreMesh(core_axis_name="c", subcore_axis_name="s")
    run = pl.kernel(
        _mae_sc_body,
        out_type=jax.ShapeDtypeStruct((NC * PROW,), jnp.float32),
        mesh=mesh,
        scratch_types=[
            pltpu.VMEM((CHUNK,), jnp.float32),
            pltpu.VMEM((CHUNK,), jnp.float32),
            pltpu.VMEM((CHUNK,), jnp.float32),
            pltpu.VMEM((CHUNK,), jnp.float32),
            pltpu.VMEM((PROW,), jnp.float32),
            pltpu.VMEM((NS * PROW,), jnp.float32),
            pltpu.VMEM((PROW,), jnp.float32),
            pltpu.VMEM_SHARED((NS * PROW,), jnp.float32),
            pltpu.SemaphoreType.DMA((2, 2)),
        ],
    )
    return run(pred_flat, true_flat)


def _mae_tc_body(p_ref, t_ref, out_ref, acc_ref, cnt_ref):
    i = pl.program_id(0)

    @pl.when(i == 0)
    def _():
        acc_ref[...] = jnp.zeros((8, LANES), jnp.float32)
        cnt_ref[...] = jnp.zeros((8, LANES), jnp.float32)

    p = p_ref[...]
    t = t_ref[...]
    m = t != 0.0
    a = jnp.where(m, jnp.abs(p - t), 0.0)
    k = jnp.where(m, 1.0, 0.0)
    acc_ref[...] += a.reshape(BR // 8, 8, LANES).sum(axis=0)
    cnt_ref[...] += k.reshape(BR // 8, 8, LANES).sum(axis=0)

    @pl.when(i == TC_GRID - 1)
    def _():
        out_ref[0, 0] = jnp.sum(acc_ref[...])
        out_ref[0, 1] = jnp.sum(cnt_ref[...])


def _mae_tc(pred2d, true2d):
    return pl.pallas_call(
        _mae_tc_body,
        grid=(TC_GRID,),
        in_specs=[
            pl.BlockSpec((BR, LANES), lambda i: (R_SC // BR + i, 0)),
            pl.BlockSpec((BR, LANES), lambda i: (R_SC // BR + i, 0)),
        ],
        out_specs=pl.BlockSpec(memory_space=pltpu.SMEM),
        out_shape=jax.ShapeDtypeStruct((1, 2), jnp.float32),
        scratch_shapes=[
            pltpu.VMEM((8, LANES), jnp.float32),
            pltpu.VMEM((8, LANES), jnp.float32),
        ],
    )(pred2d, true2d)


@jax.jit
def _mae(y_pred, y_true):
    p = y_pred.transpose(1, 2, 3, 0).reshape(ROWS, LANES)
    t = y_true.transpose(1, 2, 3, 0).reshape(ROWS, LANES)
    sc = _mae_sc(p.reshape(N), t.reshape(N))
    tc = _mae_tc(p, t)
    total = sc[0] + sc[PROW] + tc[0, 0]
    count = sc[L] + sc[PROW + L] + tc[0, 1]
    return total / count


def kernel(y_pred, y_true):
    return _mae(y_pred, y_true)


# speedup vs baseline: 1.7491x; 1.6094x over previous
import functools

import jax
import jax.numpy as jnp
from jax import lax
from jax.experimental import pallas as pl
from jax.experimental.pallas import tpu as pltpu
from jax.experimental.pallas import tpu_sc as plsc

N = 256 * 24 * 325
LANES = 128
ROWS = N // LANES

R_SC = 0
N_SC = R_SC * LANES
BR = 600
TC_GRID = (ROWS - R_SC) // BR

NC, NS, L = 2, 16, 16
NW = NC * NS
PER_W = N_SC // NW
NCHUNK = 5
CHUNK = PER_W // NCHUNK
VECS = CHUNK // L
UNROLL = 10
NACC = 4
PROW = 2 * L


def _lane_shuffle(x, idx):
    dnums = lax.GatherDimensionNumbers(
        offset_dims=(), collapsed_slice_dims=(0,), start_index_map=(0,))
    return lax.gather(x, idx[:, None], dimension_numbers=dnums,
                      slice_sizes=(1,),
                      mode=lax.GatherScatterMode.PROMISE_IN_BOUNDS)


def _lane_sum_all(x):
    iota = lax.iota(jnp.int32, L)
    for shift in (8, 4, 2, 1):
        x = x + _lane_shuffle(x, iota ^ shift)
    return x


def _mae_sc_body(pred_hbm, true_hbm, out_hbm,
                 pred0, pred1, true0, true1, partial_v, red_v, out_v,
                 shared, sems):
    cid = lax.axis_index("c")
    sid = lax.axis_index("s")
    wid = sid * NC + cid
    base = wid * PER_W
    pred_bufs = (pred0, pred1)
    true_bufs = (true0, true1)

    def start(slot, j):
        off = base + j * CHUNK
        pltpu.make_async_copy(
            pred_hbm.at[pl.ds(off, CHUNK)], pred_bufs[slot],
            sems.at[slot, 0]).start()
        pltpu.make_async_copy(
            true_hbm.at[pl.ds(off, CHUNK)], true_bufs[slot],
            sems.at[slot, 1]).start()

    def wait(slot):
        pltpu.make_async_copy(
            pred_hbm.at[pl.ds(0, CHUNK)], pred_bufs[slot],
            sems.at[slot, 0]).wait()
        pltpu.make_async_copy(
            true_hbm.at[pl.ds(0, CHUNK)], true_bufs[slot],
            sems.at[slot, 1]).wait()

    start(0, 0)
    zero = jnp.zeros((L,), jnp.float32)
    accs = (zero,) * NACC + (zero,) * NACC
    for j in range(NCHUNK):
        slot = j % 2
        if j + 1 < NCHUNK:
            start(1 - slot, j + 1)
        wait(slot)
        pv, tv = pred_bufs[slot], true_bufs[slot]

        def vec_step(i, c, pv=pv, tv=tv):
            regs = list(c)
            for u in range(UNROLL):
                r = u % NACC
                p = pv[pl.ds((i + u) * L, L)]
                t = tv[pl.ds((i + u) * L, L)]
                m = t != 0.0
                regs[r] = regs[r] + jnp.where(m, jnp.abs(p - t), 0.0)
                regs[NACC + r] = regs[NACC + r] + jnp.where(m, 1.0, 0.0)
            return tuple(regs)

        accs = plsc.parallel_loop(0, VECS, step=UNROLL, carry=accs)(vec_step)

    acc = accs[0]
    cnt = accs[NACC]
    for r in range(1, NACC):
        acc = acc + accs[r]
        cnt = cnt + accs[NACC + r]

    partial_v[pl.ds(0, L)] = acc
    partial_v[pl.ds(L, L)] = cnt
    pltpu.sync_copy(partial_v, shared.at[pl.ds(sid * PROW, PROW)])
    plsc.subcore_barrier()

    @pl.when(sid == 0)
    def _():
        pltpu.sync_copy(shared, red_v)
        acc0, cnt0 = zero, zero
        for i in range(NS):
            acc0 = acc0 + red_v[pl.ds(i * PROW, L)]
            cnt0 = cnt0 + red_v[pl.ds(i * PROW + L, L)]
        out_v[pl.ds(0, L)] = _lane_sum_all(acc0)
        out_v[pl.ds(L, L)] = _lane_sum_all(cnt0)
        pltpu.sync_copy(out_v, out_hbm.at[pl.ds(cid * PROW, PROW)])


def _mae_sc(pred_flat, true_flat):
    mesh = plsc.VectorSubcoreMesh(core_axis_name="c", subcore_axis_name="s")
    run = pl.kernel(
        _mae_sc_body,
        out_type=jax.ShapeDtypeStruct((NC * PROW,), jnp.float32),
        mesh=mesh,
        scratch_types=[
            pltpu.VMEM((CHUNK,), jnp.float32),
            pltpu.VMEM((CHUNK,), jnp.float32),
            pltpu.VMEM((CHUNK,), jnp.float32),
            pltpu.VMEM((CHUNK,), jnp.float32),
            pltpu.VMEM((PROW,), jnp.float32),
            pltpu.VMEM((NS * PROW,), jnp.float32),
            pltpu.VMEM((PROW,), jnp.float32),
            pltpu.VMEM_SHARED((NS * PROW,), jnp.float32),
            pltpu.SemaphoreType.DMA((2, 2)),
        ],
    )
    return run(pred_flat, true_flat)


def _mae_tc_body(p_ref, t_ref, out_ref, acc_ref, cnt_ref):
    i = pl.program_id(0)

    @pl.when(i == 0)
    def _():
        acc_ref[...] = jnp.zeros((8, LANES), jnp.float32)
        cnt_ref[...] = jnp.zeros((8, LANES), jnp.float32)

    p = p_ref[...]
    t = t_ref[...]
    m = t != 0.0
    a = jnp.where(m, jnp.abs(p - t), 0.0)
    k = jnp.where(m, 1.0, 0.0)
    acc_ref[...] += a.reshape(BR // 8, 8, LANES).sum(axis=0)
    cnt_ref[...] += k.reshape(BR // 8, 8, LANES).sum(axis=0)

    @pl.when(i == TC_GRID - 1)
    def _():
        out_ref[0, 0] = jnp.sum(acc_ref[...])
        out_ref[0, 1] = jnp.sum(cnt_ref[...])


def _mae_tc(pred2d, true2d):
    return pl.pallas_call(
        _mae_tc_body,
        grid=(TC_GRID,),
        in_specs=[
            pl.BlockSpec((BR, LANES), lambda i: (R_SC // BR + i, 0)),
            pl.BlockSpec((BR, LANES), lambda i: (R_SC // BR + i, 0)),
        ],
        out_specs=pl.BlockSpec(memory_space=pltpu.SMEM),
        out_shape=jax.ShapeDtypeStruct((1, 2), jnp.float32),
        scratch_shapes=[
            pltpu.VMEM((8, LANES), jnp.float32),
            pltpu.VMEM((8, LANES), jnp.float32),
        ],
    )(pred2d, true2d)


@jax.jit
def _mae(y_pred, y_true):
    p = y_pred.transpose(1, 2, 3, 0).reshape(ROWS, LANES)
    t = y_true.transpose(1, 2, 3, 0).reshape(ROWS, LANES)
    tc = _mae_tc(p, t)
    return tc[0, 0] / tc[0, 1]


def kernel(y_pred, y_true):
    return _mae(y_pred, y_true)
